# baseline (speedup 1.0000x reference)
import functools

import jax
import jax.numpy as jnp
from jax import lax
from jax.experimental import pallas as pl
from jax.experimental.pallas import tpu as pltpu
from jax.experimental.pallas import tpu_sc as plsc

N = 10000
N_NUC = 1000
E = 320000
D = 128
D_DIST = 16
H_DIM = 45

NC = 2
NS = 16
NW = NC * NS
EPW = E // NW
CH = 80
NCH = EPW // CH
ZB = 80
NZC = N // ZB

_MLP_B = 2000
_FIN_B = 2000

_TANH_C = (0.9960326307595647, -0.31214408864261367, 0.0974028508606748,
           -0.022467426780521948, 0.0035162961028893966,
           -0.0003566183293171547, 2.2266001309096044e-05,
           -7.745532673639607e-07, 1.1459663733270288e-08)


def _poly_tanh(x):
    xc = jnp.clip(x, -3.8, 3.8)
    u = xc * xc
    p = jnp.float32(_TANH_C[-1])
    for c in _TANH_C[-2::-1]:
        p = p * u + jnp.float32(c)
    return jnp.clip(xc * p, -1.0, 1.0)


def _edge_mlp_body(d_ref, w0, b0, w1, o_ref):
    h = _poly_tanh(
        jnp.dot(d_ref[...], w0[...], preferred_element_type=jnp.float32)
        + b0[...]
    )
    w = jnp.dot(h, w1[...], preferred_element_type=jnp.float32)
    o_ref[...] = pltpu.bitcast(w.astype(jnp.bfloat16), jnp.int32)


def _he_body(e_ref, w_ref, o_ref):
    o_ref[...] = jnp.dot(e_ref[...], w_ref[...],
                         preferred_element_type=jnp.float32)


def _final_body(e_ref, zs_ref, za_ref, zn_ref, gs, ga, gn, o_ref):
    acc = e_ref[...]
    acc = acc + jnp.dot(zn_ref[0] + zn_ref[1], gn[...],
                        preferred_element_type=jnp.float32)
    acc = acc + jnp.dot(zs_ref[0] + zs_ref[1], gs[...],
                        preferred_element_type=jnp.float32)
    acc = acc + jnp.dot(za_ref[0] + za_ref[1], ga[...],
                        preferred_element_type=jnp.float32)
    o_ref[...] = acc


def _sc_aggregate_body(table, we_hbm, s_hbm, r_hbm,
                       z_out,
                       sidx, ridx, rows, web, zacc,
                       sem_i, sem_g, sem_w, sem_s):
    cid = lax.axis_index("c")
    sid = lax.axis_index("s")
    wid = cid * NS + sid
    ebase = wid * EPW
    zvec = jnp.zeros((16,), jnp.float32)

    if True:
        @plsc.parallel_loop(0, ZB, 1, unroll=4)
        def _(i):
            for r in range(D // 16):
                rows[0, i, pl.ds(r * 16, 16)] = zvec

        for j in range((NZC + NS - 1) // NS):
            c = sid + NS * j

            @pl.when(c < NZC)
            def _():
                pltpu.sync_copy(rows.at[0], zacc.at[pl.ds(c * ZB, ZB)])

        plsc.subcore_barrier()

        pltpu.sync_copy(s_hbm.at[pl.ds(ebase, CH)], sidx.at[0])
        pltpu.sync_copy(r_hbm.at[pl.ds(ebase, CH)], ridx.at[0])
        pltpu.sync_copy(s_hbm.at[pl.ds(ebase + CH, CH)], sidx.at[1])
        pltpu.sync_copy(r_hbm.at[pl.ds(ebase + CH, CH)], ridx.at[1])
        pltpu.async_copy(s_hbm.at[pl.ds(ebase + 2 * CH, CH)], sidx.at[2],
                         sem_i)
        pltpu.async_copy(r_hbm.at[pl.ds(ebase + 2 * CH, CH)], ridx.at[2],
                         sem_i)
        wbase = wid * (EPW // 2)
        pltpu.async_copy(table.at[sidx.at[0]], rows.at[0], sem_g)
        pltpu.async_copy(we_hbm.at[pl.ds(wbase, CH // 2)], web.at[0], sem_w)
        pltpu.async_copy(table.at[sidx.at[1]], rows.at[1], sem_g)
        pltpu.async_copy(we_hbm.at[pl.ds(wbase + CH // 2, CH // 2)],
                         web.at[1], sem_w)

        def chunk_body(c, carry):
            p = lax.rem(c, 3)

            @pl.when(c + 3 < NCH)
            def _():
                i5 = lax.rem(c + 3, 5)
                eb = ebase + (c + 3) * CH
                pltpu.async_copy(s_hbm.at[pl.ds(eb, CH)], sidx.at[i5], sem_i)
                pltpu.async_copy(r_hbm.at[pl.ds(eb, CH)], ridx.at[i5], sem_i)

            pltpu.make_async_copy(table.at[sidx.at[lax.rem(c, 5)]],
                                  rows.at[p], sem_g).wait()
            pltpu.make_async_copy(
                we_hbm.at[pl.ds(wbase + c * (CH // 2), CH // 2)],
                web.at[p], sem_w).wait()

            @pl.when(c >= 1)
            def _():
                m = lax.rem(c - 1, 3)
                pltpu.make_async_copy(rows.at[m],
                                      zacc.at[ridx.at[lax.rem(c - 1, 5)]],
                                      sem_s).wait()

            @pl.when(c + 2 < NCH)
            def _():
                i5 = lax.rem(c + 2, 5)
                m = lax.rem(c + 2, 3)
                eb = ebase + (c + 2) * CH
                pltpu.make_async_copy(s_hbm.at[pl.ds(eb, CH)],
                                      sidx.at[i5], sem_i).wait()
                pltpu.make_async_copy(r_hbm.at[pl.ds(eb, CH)],
                                      ridx.at[i5], sem_i).wait()
                pltpu.async_copy(table.at[sidx.at[i5]], rows.at[m], sem_g)
                pltpu.async_copy(
                    we_hbm.at[pl.ds(wbase + (c + 2) * (CH // 2), CH // 2)],
                    web.at[m], sem_w)

            @plsc.parallel_loop(0, CH // 2, 1, unroll=2)
            def _(m):
                for j in range(D // 16):
                    sl = pl.ds(j * 16, 16)
                    w = web[p, m, sl]
                    lw = jax.lax.bitcast_convert_type(
                        jax.lax.shift_left(w, 16), jnp.float32)
                    hw = jax.lax.bitcast_convert_type(
                        jax.lax.bitwise_and(w, jnp.int32(-65536)),
                        jnp.float32)
                    rows[p, 2 * m, sl] = rows[p, 2 * m, sl] * lw
                    rows[p, 2 * m + 1, sl] = rows[p, 2 * m + 1, sl] * hw

            pltpu.async_copy(rows.at[p], zacc.at[ridx.at[lax.rem(c, 5)]],
                             sem_s, add=True)
            return carry

        lax.fori_loop(0, NCH, chunk_body, 0)
        pltpu.make_async_copy(rows.at[lax.rem(NCH - 1, 3)],
                              zacc.at[ridx.at[lax.rem(NCH - 1, 5)]],
                              sem_s).wait()
        plsc.subcore_barrier()

        for j in range((NZC + NS - 1) // NS):
            c = sid + NS * j

            @pl.when(c < NZC)
            def _():
                row0 = c * ZB
                pltpu.sync_copy(zacc.at[pl.ds(row0, ZB)], rows.at[0])
                pltpu.sync_copy(rows.at[0], z_out.at[cid, pl.ds(row0, ZB)])

        plsc.subcore_barrier()


def _sc_aggregate(table, we, s, r):
    mesh = plsc.VectorSubcoreMesh(core_axis_name="c", subcore_axis_name="s")
    f = functools.partial(
        pl.kernel,
        mesh=mesh,
        out_type=jax.ShapeDtypeStruct((NC, N, D), jnp.float32),
        scratch_types=[
            pltpu.VMEM((5, CH), jnp.int32),
            pltpu.VMEM((5, CH), jnp.int32),
            pltpu.VMEM((3, CH, D), jnp.float32),
            pltpu.VMEM((3, CH // 2, D), jnp.int32),
            pltpu.VMEM_SHARED((N, D), jnp.float32),
            pltpu.SemaphoreType.DMA,
            pltpu.SemaphoreType.DMA,
            pltpu.SemaphoreType.DMA,
            pltpu.SemaphoreType.DMA,
        ],
    )(_sc_aggregate_body)
    return f(table, we, s.astype(jnp.int32), r.astype(jnp.int32))


def kernel(electrons, nuclei, distances_same, distances_anti, distances_n,
           senders_same, senders_anti, senders_n,
           receivers_same, receivers_anti, receivers_n,
           w_same_W0, w_same_b0, w_same_W1,
           w_anti_W0, w_anti_b0, w_anti_W1,
           w_n_W0, w_n_b0, w_n_W1,
           h_W, g_same_W, g_anti_W, g_n_W):
    nblk = E // _MLP_B
    dist_spec = pl.BlockSpec((_MLP_B, D_DIST), lambda i: (i, 0))
    w0_spec = pl.BlockSpec((D_DIST, H_DIM), lambda i: (0, 0))
    b0_spec = pl.BlockSpec((1, H_DIM), lambda i: (0, 0))
    w1_spec = pl.BlockSpec((H_DIM, D), lambda i: (0, 0))
    we_spec = pl.BlockSpec((_MLP_B // 2, D), lambda i: (i, 0))

    def mlp(dist, w0, b0, w1):
        return pl.pallas_call(
            _edge_mlp_body,
            grid=(nblk,),
            in_specs=[dist_spec, w0_spec, b0_spec, w1_spec],
            out_specs=we_spec,
            out_shape=jax.ShapeDtypeStruct((E // 2, D), jnp.int32),
        )(dist, w0, b0.reshape(1, H_DIM), w1)

    he = pl.pallas_call(
        _he_body,
        out_shape=jax.ShapeDtypeStruct((N, D), jnp.float32),
    )(electrons, h_W)

    we_same = mlp(distances_same, w_same_W0, w_same_b0, w_same_W1)
    zs = _sc_aggregate(he, we_same, senders_same, receivers_same)
    we_anti = mlp(distances_anti, w_anti_W0, w_anti_b0, w_anti_W1)
    za = _sc_aggregate(he, we_anti, senders_anti, receivers_anti)
    we_n = mlp(distances_n, w_n_W0, w_n_b0, w_n_W1)
    zn = _sc_aggregate(nuclei, we_n, senders_n, receivers_n)

    nfin = N // _FIN_B
    row_spec = pl.BlockSpec((_FIN_B, D), lambda i: (i, 0))
    z_spec = pl.BlockSpec((NC, _FIN_B, D), lambda i: (0, i, 0))
    g_spec = pl.BlockSpec((D, D), lambda i: (0, 0))
    out = pl.pallas_call(
        _final_body,
        grid=(nfin,),
        in_specs=[row_spec, z_spec, z_spec, z_spec, g_spec, g_spec, g_spec],
        out_specs=row_spec,
        out_shape=jax.ShapeDtypeStruct((N, D), jnp.float32),
    )(electrons, zs, za, zn, g_same_W, g_anti_W, g_n_W)
    return out


# speedup vs baseline: 1.0916x; 1.0916x over previous
import functools

import jax
import jax.numpy as jnp
from jax import lax
from jax.experimental import pallas as pl
from jax.experimental.pallas import tpu as pltpu
from jax.experimental.pallas import tpu_sc as plsc

N = 10000
N_NUC = 1000
E = 320000
D = 128
D_DIST = 16
H_DIM = 45

NC = 2
NS = 16
NW = NC * NS
EPW = E // NW
CH = 80
NCH = EPW // CH
ZB = 80
NZC = N // ZB

_MLP_B = 2000
_FIN_B = 2000

def _edge_mlp_body(d_ref, w0, b0, w1, o_ref):
    h = jnp.tanh(
        jnp.dot(d_ref[...], w0[...], preferred_element_type=jnp.float32)
        + b0[...]
    )
    w = jnp.dot(h, w1[...], preferred_element_type=jnp.float32)
    o_ref[...] = pltpu.bitcast(w.astype(jnp.bfloat16), jnp.int32)


def _he_body(e_ref, w_ref, o_ref):
    o_ref[...] = jnp.dot(e_ref[...], w_ref[...],
                         preferred_element_type=jnp.float32)


def _final_body(e_ref, zs_ref, za_ref, zn_ref, gs, ga, gn, o_ref):
    acc = e_ref[...]
    acc = acc + jnp.dot(zn_ref[0] + zn_ref[1], gn[...],
                        preferred_element_type=jnp.float32)
    acc = acc + jnp.dot(zs_ref[0] + zs_ref[1], gs[...],
                        preferred_element_type=jnp.float32)
    acc = acc + jnp.dot(za_ref[0] + za_ref[1], ga[...],
                        preferred_element_type=jnp.float32)
    o_ref[...] = acc


def _sc_aggregate_body(table, we_hbm, s_hbm, r_hbm,
                       z_out,
                       sidx, ridx, rows, web, zacc,
                       sem_i, sem_g, sem_w, sem_s):
    cid = lax.axis_index("c")
    sid = lax.axis_index("s")
    wid = cid * NS + sid
    ebase = wid * EPW
    zvec = jnp.zeros((16,), jnp.float32)

    if True:
        @plsc.parallel_loop(0, ZB, 1, unroll=4)
        def _(i):
            for r in range(D // 16):
                rows[0, i, pl.ds(r * 16, 16)] = zvec

        for j in range((NZC + NS - 1) // NS):
            c = sid + NS * j

            @pl.when(c < NZC)
            def _():
                pltpu.sync_copy(rows.at[0], zacc.at[pl.ds(c * ZB, ZB)])

        plsc.subcore_barrier()

        pltpu.sync_copy(s_hbm.at[pl.ds(ebase, CH)], sidx.at[0])
        pltpu.sync_copy(r_hbm.at[pl.ds(ebase, CH)], ridx.at[0])
        pltpu.sync_copy(s_hbm.at[pl.ds(ebase + CH, CH)], sidx.at[1])
        pltpu.sync_copy(r_hbm.at[pl.ds(ebase + CH, CH)], ridx.at[1])
        pltpu.async_copy(s_hbm.at[pl.ds(ebase + 2 * CH, CH)], sidx.at[2],
                         sem_i)
        pltpu.async_copy(r_hbm.at[pl.ds(ebase + 2 * CH, CH)], ridx.at[2],
                         sem_i)
        wbase = wid * (EPW // 2)
        pltpu.async_copy(table.at[sidx.at[0]], rows.at[0], sem_g)
        pltpu.async_copy(we_hbm.at[pl.ds(wbase, CH // 2)], web.at[0], sem_w)
        pltpu.async_copy(table.at[sidx.at[1]], rows.at[1], sem_g)
        pltpu.async_copy(we_hbm.at[pl.ds(wbase + CH // 2, CH // 2)],
                         web.at[1], sem_w)

        def chunk_body(c, carry):
            p = lax.rem(c, 3)

            @pl.when(c + 3 < NCH)
            def _():
                i5 = lax.rem(c + 3, 5)
                eb = ebase + (c + 3) * CH
                pltpu.async_copy(s_hbm.at[pl.ds(eb, CH)], sidx.at[i5], sem_i)
                pltpu.async_copy(r_hbm.at[pl.ds(eb, CH)], ridx.at[i5], sem_i)

            pltpu.make_async_copy(table.at[sidx.at[lax.rem(c, 5)]],
                                  rows.at[p], sem_g).wait()
            pltpu.make_async_copy(
                we_hbm.at[pl.ds(wbase + c * (CH // 2), CH // 2)],
                web.at[p], sem_w).wait()

            @pl.when(c >= 1)
            def _():
                m = lax.rem(c - 1, 3)
                pltpu.make_async_copy(rows.at[m],
                                      zacc.at[ridx.at[lax.rem(c - 1, 5)]],
                                      sem_s).wait()

            @pl.when(c + 2 < NCH)
            def _():
                i5 = lax.rem(c + 2, 5)
                m = lax.rem(c + 2, 3)
                eb = ebase + (c + 2) * CH
                pltpu.make_async_copy(s_hbm.at[pl.ds(eb, CH)],
                                      sidx.at[i5], sem_i).wait()
                pltpu.make_async_copy(r_hbm.at[pl.ds(eb, CH)],
                                      ridx.at[i5], sem_i).wait()
                pltpu.async_copy(table.at[sidx.at[i5]], rows.at[m], sem_g)
                pltpu.async_copy(
                    we_hbm.at[pl.ds(wbase + (c + 2) * (CH // 2), CH // 2)],
                    web.at[m], sem_w)

            @plsc.parallel_loop(0, CH // 2, 1, unroll=2)
            def _(m):
                for j in range(D // 16):
                    sl = pl.ds(j * 16, 16)
                    w = web[p, m, sl]
                    lw = jax.lax.bitcast_convert_type(
                        jax.lax.shift_left(w, 16), jnp.float32)
                    hw = jax.lax.bitcast_convert_type(
                        jax.lax.bitwise_and(w, jnp.int32(-65536)),
                        jnp.float32)
                    rows[p, 2 * m, sl] = rows[p, 2 * m, sl] * lw
                    rows[p, 2 * m + 1, sl] = rows[p, 2 * m + 1, sl] * hw

            pltpu.async_copy(rows.at[p], zacc.at[ridx.at[lax.rem(c, 5)]],
                             sem_s, add=True)
            return carry

        lax.fori_loop(0, NCH, chunk_body, 0)
        pltpu.make_async_copy(rows.at[lax.rem(NCH - 1, 3)],
                              zacc.at[ridx.at[lax.rem(NCH - 1, 5)]],
                              sem_s).wait()
        plsc.subcore_barrier()

        for j in range((NZC + NS - 1) // NS):
            c = sid + NS * j

            @pl.when(c < NZC)
            def _():
                row0 = c * ZB
                pltpu.sync_copy(zacc.at[pl.ds(row0, ZB)], rows.at[0])
                pltpu.sync_copy(rows.at[0], z_out.at[cid, pl.ds(row0, ZB)])

        plsc.subcore_barrier()


def _sc_aggregate(table, we, s, r):
    mesh = plsc.VectorSubcoreMesh(core_axis_name="c", subcore_axis_name="s")
    f = functools.partial(
        pl.kernel,
        mesh=mesh,
        out_type=jax.ShapeDtypeStruct((NC, N, D), jnp.float32),
        scratch_types=[
            pltpu.VMEM((5, CH), jnp.int32),
            pltpu.VMEM((5, CH), jnp.int32),
            pltpu.VMEM((3, CH, D), jnp.float32),
            pltpu.VMEM((3, CH // 2, D), jnp.int32),
            pltpu.VMEM_SHARED((N, D), jnp.float32),
            pltpu.SemaphoreType.DMA,
            pltpu.SemaphoreType.DMA,
            pltpu.SemaphoreType.DMA,
            pltpu.SemaphoreType.DMA,
        ],
    )(_sc_aggregate_body)
    return f(table, we, s.astype(jnp.int32), r.astype(jnp.int32))


def kernel(electrons, nuclei, distances_same, distances_anti, distances_n,
           senders_same, senders_anti, senders_n,
           receivers_same, receivers_anti, receivers_n,
           w_same_W0, w_same_b0, w_same_W1,
           w_anti_W0, w_anti_b0, w_anti_W1,
           w_n_W0, w_n_b0, w_n_W1,
           h_W, g_same_W, g_anti_W, g_n_W):
    nblk = E // _MLP_B
    dist_spec = pl.BlockSpec((_MLP_B, D_DIST), lambda i: (i, 0))
    w0_spec = pl.BlockSpec((D_DIST, H_DIM), lambda i: (0, 0))
    b0_spec = pl.BlockSpec((1, H_DIM), lambda i: (0, 0))
    w1_spec = pl.BlockSpec((H_DIM, D), lambda i: (0, 0))
    we_spec = pl.BlockSpec((_MLP_B // 2, D), lambda i: (i, 0))

    def mlp(dist, w0, b0, w1):
        return pl.pallas_call(
            _edge_mlp_body,
            grid=(nblk,),
            in_specs=[dist_spec, w0_spec, b0_spec, w1_spec],
            out_specs=we_spec,
            out_shape=jax.ShapeDtypeStruct((E // 2, D), jnp.int32),
        )(dist, w0, b0.reshape(1, H_DIM), w1)

    he = pl.pallas_call(
        _he_body,
        out_shape=jax.ShapeDtypeStruct((N, D), jnp.float32),
    )(electrons, h_W)

    we_same = mlp(distances_same, w_same_W0, w_same_b0, w_same_W1)
    zs = _sc_aggregate(he, we_same, senders_same, receivers_same)
    we_anti = mlp(distances_anti, w_anti_W0, w_anti_b0, w_anti_W1)
    za = _sc_aggregate(he, we_anti, senders_anti, receivers_anti)
    we_n = mlp(distances_n, w_n_W0, w_n_b0, w_n_W1)
    zn = _sc_aggregate(nuclei, we_n, senders_n, receivers_n)

    nfin = N // _FIN_B
    row_spec = pl.BlockSpec((_FIN_B, D), lambda i: (i, 0))
    z_spec = pl.BlockSpec((NC, _FIN_B, D), lambda i: (0, i, 0))
    g_spec = pl.BlockSpec((D, D), lambda i: (0, 0))
    out = pl.pallas_call(
        _final_body,
        grid=(nfin,),
        in_specs=[row_spec, z_spec, z_spec, z_spec, g_spec, g_spec, g_spec],
        out_specs=row_spec,
        out_shape=jax.ShapeDtypeStruct((N, D), jnp.float32),
    )(electrons, zs, za, zn, g_same_W, g_anti_W, g_n_W)
    return out
